# Initial kernel scaffold; baseline (speedup 1.0000x reference)
import jax
import jax.numpy as jnp
from jax.experimental import pallas as pl


def kernel(values, C, rows, cols):
    raise NotImplementedError("write your pallas kernel here")


# speedup vs baseline: 21.0251x; 21.0251x over previous
import jax
import jax.numpy as jnp
from jax import lax
from jax.experimental import pallas as pl
from jax.experimental.pallas import tpu as pltpu
from jax.experimental.pallas import tpu_sc as plsc

N_ORB = 4096
S_TOTAL = 262144
NW = 32
SPT = S_TOTAL // NW
NSTRIP = 16
SR = N_ORB // NSTRIP
STRIP_ELEMS = SR * N_ORB
STRIP_PAD = 64
NB = 64
SCAN_VREGS = SPT // 16


def _sc_body(vals16_hbm, c2_hbm, rows_hbm, cols_hbm, out_hbm,
             rows_v, cols_v, c2_v, sel_v, gidx_v, vals_g, st_idx, st_upd,
             zbuf, strip, sem):
    cid = lax.axis_index("c")
    sid = lax.axis_index("s")
    wid = cid * 16 + sid
    slice16 = STRIP_ELEMS // 16

    pltpu.sync_copy(rows_hbm.at[pl.ds(wid * SPT, SPT)], rows_v.at[pl.ds(0, SPT)])
    pltpu.sync_copy(cols_hbm.at[pl.ds(wid * SPT, SPT)], cols_v.at[pl.ds(0, SPT)])
    pltpu.sync_copy(c2_hbm, c2_v)
    rows_v[pl.ds(SPT, 16)] = jnp.full((16,), 1 << 20, dtype=jnp.int32)
    cols_v[pl.ds(SPT, 16)] = jnp.zeros((16,), dtype=jnp.int32)

    def _zero_zbuf(i, carry):
        zbuf[pl.ds(i * 16, 16)] = jnp.zeros((16,), dtype=jnp.float32)
        return carry
    lax.fori_loop(0, 4096 // 16, _zero_zbuf, 0)

    lanes = lax.iota(jnp.int32, 16)
    qv = [lanes + k * 16 for k in range(4)]
    qd = [jnp.where(q < 49, q // 7, 100000) for q in qv]
    qm = [jnp.where(q < 49, q % 7, 0) for q in qv]
    dumpv = [STRIP_ELEMS + q for q in qv]

    def _strip_pass(p, carry):
        for j in range(16):
            pltpu.sync_copy(zbuf, strip.at[pl.ds(sid * slice16 + j * 4096, 4096)])
        plsc.subcore_barrier()

        def _scan(k, cnt):
            r16 = rows_v[pl.ds(k * 16, 16)]
            p0 = lax.shift_right_logical(r16, 8)
            p1 = lax.shift_right_logical(r16 + 6, 8)
            m = (p0 == p) | (p1 == p)
            ids = lanes + k * 16
            plsc.store_compressed(sel_v.at[pl.ds(cnt, 16)], ids, mask=m)
            return cnt + jnp.max(plsc.all_reduce_population_count(m))
        cnt = lax.fori_loop(0, SCAN_VREGS, _scan, jnp.int32(0))
        sent = jnp.full((16,), SPT, dtype=jnp.int32)
        for t in range(NB // 16):
            sel_v[pl.ds(cnt + t * 16, 16)] = sent

        def _batch(b, carry):
            for t in range(NB // 16):
                sl = sel_v[pl.ds(b * NB + t * 16, 16)]
                gid = jnp.minimum(sl + wid * SPT, S_TOTAL - 1)
                gidx_v[pl.ds(t * 16, 16)] = gid
            pltpu.async_copy(vals16_hbm.at[gidx_v], vals_g, sem).wait()

            def _sample(i, carry):
                s = sel_v[pl.ds(b * NB + i, 16)][0]
                r = rows_v[pl.ds(s, 16)][0]
                c = cols_v[pl.ds(s, 16)][0]
                v16 = vals_g[i]
                rowoff = r - p * SR
                acc = [jnp.zeros((16,), dtype=jnp.float32) for _ in range(4)]
                for mm in range(7):
                    vb = jnp.broadcast_to(v16[mm], (16,))
                    for k in range(4):
                        acc[k] = acc[k] + c2_v[pl.ds(mm * 64 + k * 16, 16)] * vb
                cb = jnp.broadcast_to(c, (16,))
                rb = jnp.broadcast_to(rowoff, (16,))
                for k in range(4):
                    ro = rb + qd[k]
                    valid = (ro >= 0) & (ro < SR)
                    idx = lax.shift_left(ro, 12) + cb + qm[k]
                    st_idx[pl.ds(i * 64 + k * 16, 16)] = jnp.where(valid, idx, dumpv[k])
                    st_upd[pl.ds(i * 64 + k * 16, 16)] = jnp.where(valid, acc[k], 0.0)
                return carry
            lax.fori_loop(0, NB, _sample, 0)
            pltpu.sync_copy(st_upd, strip.at[st_idx], add=True)
            return carry
        nb = (cnt + (NB - 1)) // NB
        lax.fori_loop(0, nb, _batch, 0)
        plsc.subcore_barrier()

        out_off = cid * (NSTRIP * STRIP_ELEMS) + p * STRIP_ELEMS + sid * slice16
        pltpu.sync_copy(strip.at[pl.ds(sid * slice16, slice16)],
                        out_hbm.at[pl.ds(out_off, slice16)])
        plsc.subcore_barrier()
        return carry

    lax.fori_loop(0, NSTRIP, _strip_pass, 0)


def _scatter_partials(vals16, c2_flat, rows, cols):
    mesh = plsc.VectorSubcoreMesh(core_axis_name="c", subcore_axis_name="s")
    return pl.kernel(
        _sc_body,
        out_type=jax.ShapeDtypeStruct((2 * NSTRIP * STRIP_ELEMS,), jnp.float32),
        mesh=mesh,
        scratch_types=[
            pltpu.VMEM((SPT + 16,), jnp.int32),
            pltpu.VMEM((SPT + 16,), jnp.int32),
            pltpu.VMEM((448,), jnp.float32),
            pltpu.VMEM((SPT + NB,), jnp.int32),
            pltpu.VMEM((NB,), jnp.int32),
            pltpu.VMEM((NB, 16), jnp.float32),
            pltpu.VMEM((NB * 64,), jnp.int32),
            pltpu.VMEM((NB * 64,), jnp.float32),
            pltpu.VMEM((4096,), jnp.float32),
            pltpu.VMEM_SHARED((STRIP_ELEMS + STRIP_PAD,), jnp.float32),
            pltpu.SemaphoreType.DMA,
        ],
        compiler_params=pltpu.CompilerParams(needs_layout_passes=False,
                                             use_tc_tiling_on_sc=False),
    )(vals16, c2_flat, rows, cols)


def _sym_body(a0, a1, b0, b1, o):
    b = b0[...] + b1[...]
    o[...] = 0.5 * (a0[...] + a1[...] + b.T)


def _symmetrize(A0, A1):
    blk = 256
    g = N_ORB // blk
    return pl.pallas_call(
        _sym_body,
        grid=(g, g),
        in_specs=[
            pl.BlockSpec((blk, blk), lambda i, j: (i, j)),
            pl.BlockSpec((blk, blk), lambda i, j: (i, j)),
            pl.BlockSpec((blk, blk), lambda i, j: (j, i)),
            pl.BlockSpec((blk, blk), lambda i, j: (j, i)),
        ],
        out_specs=pl.BlockSpec((blk, blk), lambda i, j: (i, j)),
        out_shape=jax.ShapeDtypeStruct((N_ORB, N_ORB), jnp.float32),
    )(A0, A1, A0, A1)


@jax.jit
def kernel(values, C, rows, cols):
    c2 = jnp.zeros((7, 64), dtype=jnp.float32)
    c2 = c2.at[:, :49].set(jnp.transpose(C, (2, 0, 1)).reshape(7, 49))
    vals16 = jnp.zeros((S_TOTAL, 16), dtype=jnp.float32)
    vals16 = vals16.at[:, :7].set(values)
    partials = _scatter_partials(vals16, c2.reshape(-1),
                                 rows.astype(jnp.int32), cols.astype(jnp.int32))
    A = partials.reshape(2, N_ORB, N_ORB)
    return _symmetrize(A[0], A[1])


# speedup vs baseline: 23.9056x; 1.1370x over previous
import jax
import jax.numpy as jnp
from jax import lax
from jax.experimental import pallas as pl
from jax.experimental.pallas import tpu as pltpu
from jax.experimental.pallas import tpu_sc as plsc

N_ORB = 4096
S_TOTAL = 262144
NW = 32
SPT = S_TOTAL // NW
NSTRIP = 16
SR = N_ORB // NSTRIP
STRIP_ELEMS = SR * N_ORB
STRIP_PAD = 64
NB = 32
NG = NB // 16
SCAN_VREGS = SPT // 16


def _sc_body(vals_hbm, c2_hbm, rows_hbm, cols_hbm, out_hbm,
             rows_v, cols_v, c2_v, sel_v, gidx_v, vals_g, st_idx, st_upd,
             zbuf, strip, sem):
    cid = lax.axis_index("c")
    sid = lax.axis_index("s")
    wid = cid * 16 + sid
    slice16 = STRIP_ELEMS // 16

    pltpu.sync_copy(rows_hbm.at[pl.ds(wid * SPT, SPT)], rows_v.at[pl.ds(0, SPT)])
    pltpu.sync_copy(cols_hbm.at[pl.ds(wid * SPT, SPT)], cols_v.at[pl.ds(0, SPT)])
    pltpu.sync_copy(c2_hbm, c2_v)
    rows_v[pl.ds(SPT, 16)] = jnp.full((16,), 1 << 20, dtype=jnp.int32)
    cols_v[pl.ds(SPT, 16)] = jnp.zeros((16,), dtype=jnp.int32)

    def _zero_zbuf(i, carry):
        zbuf[pl.ds(i * 16, 16)] = jnp.zeros((16,), dtype=jnp.float32)
        return carry
    lax.fori_loop(0, 4096 // 16, _zero_zbuf, 0)

    lanes = lax.iota(jnp.int32, 16)
    dump16 = STRIP_ELEMS + lanes

    def _strip_pass(p, carry):
        for j in range(16):
            pltpu.sync_copy(zbuf, strip.at[pl.ds(sid * slice16 + j * 4096, 4096)])
        plsc.subcore_barrier()

        def _scan(k, cnt):
            r16 = rows_v[pl.ds(k * 16, 16)]
            p0 = lax.shift_right_logical(r16, 8)
            p1 = lax.shift_right_logical(r16 + 6, 8)
            m = (p0 == p) | (p1 == p)
            ids = lanes + k * 16
            plsc.store_compressed(sel_v.at[pl.ds(cnt, 16)], ids, mask=m)
            return cnt + jnp.max(plsc.all_reduce_population_count(m))
        cnt = lax.fori_loop(0, SCAN_VREGS, _scan, jnp.int32(0))
        sent = jnp.full((16,), SPT, dtype=jnp.int32)
        for t in range(NG):
            sel_v[pl.ds(cnt + t * 16, 16)] = sent

        def _batch(b, carry):
            s16s = []
            for t in range(NG):
                s16 = sel_v[pl.ds((b * NG + t) * 16, 16)]
                s16s.append(s16)
                gid = jnp.minimum(s16 + wid * SPT, S_TOTAL - 1)
                for mq in range(7):
                    gidx_v[pl.ds(mq * NB + t * 16, 16)] = gid * 7 + mq
            pltpu.async_copy(vals_hbm.at[gidx_v], vals_g, sem).wait()

            c2vecs = [[c2_v[pl.ds(mq * 64 + k * 16, 16)] for mq in range(7)]
                      for k in range(4)]
            for t in range(NG):
                s16 = s16s[t]
                r16 = plsc.load_gather(rows_v, [s16])
                c16 = plsc.load_gather(cols_v, [s16])
                ro16 = r16 - p * SR
                base16 = lax.shift_left(ro16, 12) + c16
                xm = [vals_g[pl.ds(mq * NB + t * 16, 16)] for mq in range(7)]
                for q in range(49):
                    k, lq = q // 16, q % 16
                    lqv = jnp.full((16,), lq, dtype=jnp.int32)
                    acc = xm[0] * jnp.take_along_axis(
                        c2vecs[k][0], lqv, axis=0, mode="promise_in_bounds")
                    for mq in range(1, 7):
                        acc = acc + xm[mq] * jnp.take_along_axis(
                            c2vecs[k][mq], lqv, axis=0, mode="promise_in_bounds")
                    roq = ro16 + (q // 7)
                    valid = (roq >= 0) & (roq < SR)
                    idxq = base16 + ((q // 7) * N_ORB + q % 7)
                    off = (t * 49 + q) * 16
                    st_idx[pl.ds(off, 16)] = jnp.where(valid, idxq, dump16)
                    st_upd[pl.ds(off, 16)] = jnp.where(valid, acc, 0.0)
            pltpu.sync_copy(st_upd, strip.at[st_idx], add=True)
            return carry
        nb = (cnt + (NB - 1)) // NB
        lax.fori_loop(0, nb, _batch, 0)
        plsc.subcore_barrier()

        out_off = cid * (NSTRIP * STRIP_ELEMS) + p * STRIP_ELEMS + sid * slice16
        pltpu.sync_copy(strip.at[pl.ds(sid * slice16, slice16)],
                        out_hbm.at[pl.ds(out_off, slice16)])
        plsc.subcore_barrier()
        return carry

    lax.fori_loop(0, NSTRIP, _strip_pass, 0)


def _scatter_partials(vals_flat, c2_flat, rows, cols):
    mesh = plsc.VectorSubcoreMesh(core_axis_name="c", subcore_axis_name="s")
    return pl.kernel(
        _sc_body,
        out_type=jax.ShapeDtypeStruct((2 * NSTRIP * STRIP_ELEMS,), jnp.float32),
        mesh=mesh,
        scratch_types=[
            pltpu.VMEM((SPT + 16,), jnp.int32),
            pltpu.VMEM((SPT + 16,), jnp.int32),
            pltpu.VMEM((448,), jnp.float32),
            pltpu.VMEM((SPT + NB,), jnp.int32),
            pltpu.VMEM((7 * NB,), jnp.int32),
            pltpu.VMEM((7 * NB,), jnp.float32),
            pltpu.VMEM((NG * 49 * 16,), jnp.int32),
            pltpu.VMEM((NG * 49 * 16,), jnp.float32),
            pltpu.VMEM((4096,), jnp.float32),
            pltpu.VMEM_SHARED((STRIP_ELEMS + STRIP_PAD,), jnp.float32),
            pltpu.SemaphoreType.DMA,
        ],
        compiler_params=pltpu.CompilerParams(needs_layout_passes=False,
                                             use_tc_tiling_on_sc=False),
    )(vals_flat, c2_flat, rows, cols)


def _sym_body(a0, a1, b0, b1, o):
    b = b0[0] + b1[0]
    o[...] = 0.5 * (a0[0] + a1[0] + b.T)


def _symmetrize(A):
    blk = 256
    g = N_ORB // blk
    return pl.pallas_call(
        _sym_body,
        grid=(g, g),
        in_specs=[
            pl.BlockSpec((1, blk, blk), lambda i, j: (0, i, j)),
            pl.BlockSpec((1, blk, blk), lambda i, j: (1, i, j)),
            pl.BlockSpec((1, blk, blk), lambda i, j: (0, j, i)),
            pl.BlockSpec((1, blk, blk), lambda i, j: (1, j, i)),
        ],
        out_specs=pl.BlockSpec((blk, blk), lambda i, j: (i, j)),
        out_shape=jax.ShapeDtypeStruct((N_ORB, N_ORB), jnp.float32),
    )(A, A, A, A)


@jax.jit
def kernel(values, C, rows, cols):
    c2 = jnp.zeros((7, 64), dtype=jnp.float32)
    c2 = c2.at[:, :49].set(jnp.transpose(C, (2, 0, 1)).reshape(7, 49))
    partials = _scatter_partials(values.reshape(-1), c2.reshape(-1),
                                 rows.astype(jnp.int32), cols.astype(jnp.int32))
    return _symmetrize(partials.reshape(2, N_ORB, N_ORB))


# speedup vs baseline: 25.5804x; 1.0701x over previous
import jax
import jax.numpy as jnp
from jax import lax
from jax.experimental import pallas as pl
from jax.experimental.pallas import tpu as pltpu
from jax.experimental.pallas import tpu_sc as plsc

N_ORB = 4096
S_TOTAL = 262144
NW = 32
SPT = S_TOTAL // NW
NSTRIP = 16
SR = N_ORB // NSTRIP
STRIP_ELEMS = SR * N_ORB
STRIP_PAD = 64
NB = 64
NG = NB // 16
SCAN_VREGS = SPT // 16


def _sc_body(vals_hbm, c2_hbm, rows_hbm, cols_hbm, out_hbm,
             rows_v, cols_v, c2_v, sel_v, gidx_v, vals_g, st_idx, st_upd,
             zbuf, strip, sem):
    cid = lax.axis_index("c")
    sid = lax.axis_index("s")
    wid = cid * 16 + sid
    slice16 = STRIP_ELEMS // 16

    pltpu.sync_copy(rows_hbm.at[pl.ds(wid * SPT, SPT)], rows_v.at[pl.ds(0, SPT)])
    pltpu.sync_copy(cols_hbm.at[pl.ds(wid * SPT, SPT)], cols_v.at[pl.ds(0, SPT)])
    pltpu.sync_copy(c2_hbm, c2_v)
    rows_v[pl.ds(SPT, 16)] = jnp.full((16,), 1 << 20, dtype=jnp.int32)
    cols_v[pl.ds(SPT, 16)] = jnp.zeros((16,), dtype=jnp.int32)

    def _zero_zbuf(i, carry):
        zbuf[pl.ds(i * 16, 16)] = jnp.zeros((16,), dtype=jnp.float32)
        return carry
    lax.fori_loop(0, 4096 // 16, _zero_zbuf, 0)

    lanes = lax.iota(jnp.int32, 16)
    dump16 = STRIP_ELEMS + lanes

    def _strip_pass(p, carry):
        for j in range(16):
            pltpu.sync_copy(zbuf, strip.at[pl.ds(sid * slice16 + j * 4096, 4096)])
        plsc.subcore_barrier()

        def _scan(k, cnt):
            r16 = rows_v[pl.ds(k * 16, 16)]
            p0 = lax.shift_right_logical(r16, 8)
            p1 = lax.shift_right_logical(r16 + 6, 8)
            m = (p0 == p) | (p1 == p)
            ids = lanes + k * 16
            plsc.store_compressed(sel_v.at[pl.ds(cnt, 16)], ids, mask=m)
            return cnt + plsc.all_reduce_population_count(m)[0]
        cnt = lax.fori_loop(0, SCAN_VREGS, _scan, jnp.int32(0))
        sent = jnp.full((16,), SPT, dtype=jnp.int32)
        for t in range(NG):
            sel_v[pl.ds(cnt + t * 16, 16)] = sent

        def _batch(b, carry):
            s16s = []
            for t in range(NG):
                s16 = sel_v[pl.ds((b * NG + t) * 16, 16)]
                s16s.append(s16)
                gid = jnp.minimum(s16 + wid * SPT, S_TOTAL - 1)
                for mq in range(7):
                    gidx_v[pl.ds(mq * NB + t * 16, 16)] = gid * 7 + mq
            pltpu.async_copy(vals_hbm.at[gidx_v], vals_g, sem).wait()

            for t in range(NG):
                s16 = s16s[t]
                r16 = plsc.load_gather(rows_v, [s16])
                c16 = plsc.load_gather(cols_v, [s16])
                ro16 = r16 - p * SR
                base16 = lax.shift_left(ro16, 12) + c16
                xm = [vals_g[pl.ds(mq * NB + t * 16, 16)] for mq in range(7)]
                valids = []
                idxb = []
                for qd in range(7):
                    roq = ro16 + qd
                    valids.append((roq >= 0) & (roq < SR))
                    idxb.append(base16 + qd * N_ORB)
                for q in range(49):
                    k, lq = q // 16, q % 16
                    lqv = jnp.full((16,), lq, dtype=jnp.int32)
                    acc = xm[0] * jnp.take_along_axis(
                        c2_v[pl.ds(0 * 64 + k * 16, 16)], lqv, axis=0,
                        mode="promise_in_bounds")
                    for mq in range(1, 7):
                        acc = acc + xm[mq] * jnp.take_along_axis(
                            c2_v[pl.ds(mq * 64 + k * 16, 16)], lqv, axis=0,
                            mode="promise_in_bounds")
                    valid = valids[q // 7]
                    idxq = idxb[q // 7] + (q % 7)
                    off = (t * 49 + q) * 16
                    st_idx[pl.ds(off, 16)] = jnp.where(valid, idxq, dump16)
                    st_upd[pl.ds(off, 16)] = jnp.where(valid, acc, 0.0)
            pltpu.sync_copy(st_upd, strip.at[st_idx], add=True)
            return carry
        nb = (cnt + (NB - 1)) // NB
        lax.fori_loop(0, nb, _batch, 0)
        plsc.subcore_barrier()

        out_off = cid * (NSTRIP * STRIP_ELEMS) + p * STRIP_ELEMS + sid * slice16
        pltpu.sync_copy(strip.at[pl.ds(sid * slice16, slice16)],
                        out_hbm.at[pl.ds(out_off, slice16)])
        plsc.subcore_barrier()
        return carry

    lax.fori_loop(0, NSTRIP, _strip_pass, 0)


def _scatter_partials(vals_flat, c2_flat, rows, cols):
    mesh = plsc.VectorSubcoreMesh(core_axis_name="c", subcore_axis_name="s")
    return pl.kernel(
        _sc_body,
        out_type=jax.ShapeDtypeStruct((2 * NSTRIP * STRIP_ELEMS,), jnp.float32),
        mesh=mesh,
        scratch_types=[
            pltpu.VMEM((SPT + 16,), jnp.int32),
            pltpu.VMEM((SPT + 16,), jnp.int32),
            pltpu.VMEM((448,), jnp.float32),
            pltpu.VMEM((SPT + NB,), jnp.int32),
            pltpu.VMEM((7 * NB,), jnp.int32),
            pltpu.VMEM((7 * NB,), jnp.float32),
            pltpu.VMEM((NG * 49 * 16,), jnp.int32),
            pltpu.VMEM((NG * 49 * 16,), jnp.float32),
            pltpu.VMEM((4096,), jnp.float32),
            pltpu.VMEM_SHARED((STRIP_ELEMS + STRIP_PAD,), jnp.float32),
            pltpu.SemaphoreType.DMA,
        ],
        compiler_params=pltpu.CompilerParams(needs_layout_passes=False,
                                             use_tc_tiling_on_sc=False),
    )(vals_flat, c2_flat, rows, cols)


def _sym_body(a0, a1, b0, b1, o):
    b = b0[0] + b1[0]
    o[...] = 0.5 * (a0[0] + a1[0] + b.T)


def _symmetrize(A):
    blk = 256
    g = N_ORB // blk
    return pl.pallas_call(
        _sym_body,
        grid=(g, g),
        in_specs=[
            pl.BlockSpec((1, blk, blk), lambda i, j: (0, i, j)),
            pl.BlockSpec((1, blk, blk), lambda i, j: (1, i, j)),
            pl.BlockSpec((1, blk, blk), lambda i, j: (0, j, i)),
            pl.BlockSpec((1, blk, blk), lambda i, j: (1, j, i)),
        ],
        out_specs=pl.BlockSpec((blk, blk), lambda i, j: (i, j)),
        out_shape=jax.ShapeDtypeStruct((N_ORB, N_ORB), jnp.float32),
    )(A, A, A, A)


@jax.jit
def kernel(values, C, rows, cols):
    c2 = jnp.zeros((7, 64), dtype=jnp.float32)
    c2 = c2.at[:, :49].set(jnp.transpose(C, (2, 0, 1)).reshape(7, 49))
    partials = _scatter_partials(values.reshape(-1), c2.reshape(-1),
                                 rows.astype(jnp.int32), cols.astype(jnp.int32))
    return _symmetrize(partials.reshape(2, N_ORB, N_ORB))


# speedup vs baseline: 28.0797x; 1.0977x over previous
import jax
import jax.numpy as jnp
from jax import lax
from jax.experimental import pallas as pl
from jax.experimental.pallas import tpu as pltpu
from jax.experimental.pallas import tpu_sc as plsc

N_ORB = 4096
S_TOTAL = 262144
NW = 32
SPT = S_TOTAL // NW
NSTRIP = 16
SR = N_ORB // NSTRIP
STRIP_ELEMS = SR * N_ORB
STRIP_PAD = 64
NB = 64
NG = NB // 16
SCAN_VREGS = SPT // 16


def _sc_body(vals_hbm, c2_hbm, rows_hbm, cols_hbm, out_hbm,
             rows_v, cols_v, c2_v, sel_v, gidx_v, vals_g, st_idx, st_upd,
             zbuf, strip, sem, sem2):
    cid = lax.axis_index("c")
    sid = lax.axis_index("s")
    wid = cid * 16 + sid
    slice16 = STRIP_ELEMS // 16

    pltpu.sync_copy(rows_hbm.at[pl.ds(wid * SPT, SPT)], rows_v.at[pl.ds(0, SPT)])
    pltpu.sync_copy(cols_hbm.at[pl.ds(wid * SPT, SPT)], cols_v.at[pl.ds(0, SPT)])
    pltpu.sync_copy(c2_hbm, c2_v)
    rows_v[pl.ds(SPT, 16)] = jnp.full((16,), 1 << 20, dtype=jnp.int32)
    cols_v[pl.ds(SPT, 16)] = jnp.zeros((16,), dtype=jnp.int32)

    def _zero_zbuf(i, carry):
        zbuf[pl.ds(i * 16, 16)] = jnp.zeros((16,), dtype=jnp.float32)
        return carry
    lax.fori_loop(0, 4096 // 16, _zero_zbuf, 0)

    lanes = lax.iota(jnp.int32, 16)
    dump16 = STRIP_ELEMS + lanes

    def _strip_pass(p, carry):
        for j in range(16):
            pltpu.sync_copy(zbuf, strip.at[pl.ds(sid * slice16 + j * 4096, 4096)])
        plsc.subcore_barrier()

        def _scan(k, cnt):
            r16 = rows_v[pl.ds(k * 16, 16)]
            p0 = lax.shift_right_logical(r16, 8)
            p1 = lax.shift_right_logical(r16 + 6, 8)
            m = (p0 == p) | (p1 == p)
            ids = lanes + k * 16
            plsc.store_compressed(sel_v.at[pl.ds(cnt, 16)], ids, mask=m)
            return cnt + plsc.all_reduce_population_count(m)[0]
        cnt = lax.fori_loop(0, SCAN_VREGS, _scan, jnp.int32(0))
        sent = jnp.full((16,), SPT, dtype=jnp.int32)
        for t in range(NG):
            sel_v[pl.ds(cnt + t * 16, 16)] = sent

        def _batch(b, carry):
            s16s = []
            for t in range(NG):
                s16 = sel_v[pl.ds((b * NG + t) * 16, 16)]
                s16s.append(s16)
                gid = jnp.minimum(s16 + wid * SPT, S_TOTAL - 1)
                for mq in range(7):
                    gidx_v[pl.ds(mq * NB + t * 16, 16)] = gid * 7 + mq
            pltpu.async_copy(vals_hbm.at[gidx_v], vals_g, sem).wait()

            @pl.when(b > 0)
            def _():
                pltpu.make_async_copy(st_upd, strip.at[st_idx], sem2).wait()

            for t in range(NG):
                s16 = s16s[t]
                r16 = plsc.load_gather(rows_v, [s16])
                c16 = plsc.load_gather(cols_v, [s16])
                ro16 = r16 - p * SR
                base16 = lax.shift_left(ro16, 12) + c16
                xm = [vals_g[pl.ds(mq * NB + t * 16, 16)] for mq in range(7)]
                valids = []
                idxb = []
                for qd in range(7):
                    roq = ro16 + qd
                    valids.append((roq >= 0) & (roq < SR))
                    idxb.append(base16 + qd * N_ORB)
                for q in range(49):
                    k, lq = q // 16, q % 16
                    lqv = jnp.full((16,), lq, dtype=jnp.int32)
                    pr = [xm[mq] * jnp.take_along_axis(
                              c2_v[pl.ds(mq * 64 + k * 16, 16)], lqv, axis=0,
                              mode="promise_in_bounds")
                          for mq in range(7)]
                    acc = ((pr[0] + pr[1]) + (pr[2] + pr[3])) + (
                        (pr[4] + pr[5]) + pr[6])
                    valid = valids[q // 7]
                    idxq = idxb[q // 7] + (q % 7)
                    off = (t * 49 + q) * 16
                    st_idx[pl.ds(off, 16)] = jnp.where(valid, idxq, dump16)
                    st_upd[pl.ds(off, 16)] = jnp.where(valid, acc, 0.0)
            pltpu.async_copy(st_upd, strip.at[st_idx], sem2, add=True)
            return carry
        nb = (cnt + (NB - 1)) // NB
        lax.fori_loop(0, nb, _batch, 0)

        @pl.when(nb > 0)
        def _():
            pltpu.make_async_copy(st_upd, strip.at[st_idx], sem2).wait()
        plsc.subcore_barrier()

        out_off = cid * (NSTRIP * STRIP_ELEMS) + p * STRIP_ELEMS + sid * slice16
        pltpu.sync_copy(strip.at[pl.ds(sid * slice16, slice16)],
                        out_hbm.at[pl.ds(out_off, slice16)])
        plsc.subcore_barrier()
        return carry

    lax.fori_loop(0, NSTRIP, _strip_pass, 0)


def _scatter_partials(vals_flat, c2_flat, rows, cols):
    mesh = plsc.VectorSubcoreMesh(core_axis_name="c", subcore_axis_name="s")
    return pl.kernel(
        _sc_body,
        out_type=jax.ShapeDtypeStruct((2 * NSTRIP * STRIP_ELEMS,), jnp.float32),
        mesh=mesh,
        scratch_types=[
            pltpu.VMEM((SPT + 16,), jnp.int32),
            pltpu.VMEM((SPT + 16,), jnp.int32),
            pltpu.VMEM((448,), jnp.float32),
            pltpu.VMEM((SPT + NB,), jnp.int32),
            pltpu.VMEM((7 * NB,), jnp.int32),
            pltpu.VMEM((7 * NB,), jnp.float32),
            pltpu.VMEM((NG * 49 * 16,), jnp.int32),
            pltpu.VMEM((NG * 49 * 16,), jnp.float32),
            pltpu.VMEM((4096,), jnp.float32),
            pltpu.VMEM_SHARED((STRIP_ELEMS + STRIP_PAD,), jnp.float32),
            pltpu.SemaphoreType.DMA,
            pltpu.SemaphoreType.DMA,
        ],
        compiler_params=pltpu.CompilerParams(needs_layout_passes=False,
                                             use_tc_tiling_on_sc=False),
    )(vals_flat, c2_flat, rows, cols)


def _sym_body(a0, a1, b0, b1, o):
    b = b0[0] + b1[0]
    o[...] = 0.5 * (a0[0] + a1[0] + b.T)


def _symmetrize(A):
    blk = 256
    g = N_ORB // blk
    return pl.pallas_call(
        _sym_body,
        grid=(g, g),
        in_specs=[
            pl.BlockSpec((1, blk, blk), lambda i, j: (0, i, j)),
            pl.BlockSpec((1, blk, blk), lambda i, j: (1, i, j)),
            pl.BlockSpec((1, blk, blk), lambda i, j: (0, j, i)),
            pl.BlockSpec((1, blk, blk), lambda i, j: (1, j, i)),
        ],
        out_specs=pl.BlockSpec((blk, blk), lambda i, j: (i, j)),
        out_shape=jax.ShapeDtypeStruct((N_ORB, N_ORB), jnp.float32),
    )(A, A, A, A)


@jax.jit
def kernel(values, C, rows, cols):
    c2 = jnp.zeros((7, 64), dtype=jnp.float32)
    c2 = c2.at[:, :49].set(jnp.transpose(C, (2, 0, 1)).reshape(7, 49))
    partials = _scatter_partials(values.reshape(-1), c2.reshape(-1),
                                 rows.astype(jnp.int32), cols.astype(jnp.int32))
    return _symmetrize(partials.reshape(2, N_ORB, N_ORB))


# speedup vs baseline: 30.4467x; 1.0843x over previous
import jax
import jax.numpy as jnp
from jax import lax
from jax.experimental import pallas as pl
from jax.experimental.pallas import tpu as pltpu
from jax.experimental.pallas import tpu_sc as plsc

N_ORB = 4096
S_TOTAL = 262144
NW = 32
SPT = S_TOTAL // NW
NSTRIP = 16
SR = N_ORB // NSTRIP
STRIP_ELEMS = SR * N_ORB
STRIP_PAD = 64
NB = 64
NG = NB // 16
SCAN_VREGS = SPT // 16


def _sc_body(vals_hbm, c2_hbm, rows_hbm, cols_hbm, out_hbm,
             rows_v, cols_v, c2_v, sel_v, gidx_v, vals_g, st_idx, st_upd,
             zbuf, strip, sem, sem2):
    cid = lax.axis_index("c")
    sid = lax.axis_index("s")
    wid = cid * 16 + sid
    slice16 = STRIP_ELEMS // 16

    pltpu.sync_copy(rows_hbm.at[pl.ds(wid * SPT, SPT)], rows_v.at[pl.ds(0, SPT)])
    pltpu.sync_copy(cols_hbm.at[pl.ds(wid * SPT, SPT)], cols_v.at[pl.ds(0, SPT)])
    pltpu.sync_copy(c2_hbm, c2_v)
    rows_v[pl.ds(SPT, 16)] = jnp.full((16,), 1 << 20, dtype=jnp.int32)
    cols_v[pl.ds(SPT, 16)] = jnp.zeros((16,), dtype=jnp.int32)

    def _zero_zbuf(i, carry):
        zbuf[pl.ds(i * 16, 16)] = jnp.zeros((16,), dtype=jnp.float32)
        return carry
    lax.fori_loop(0, 4096 // 16, _zero_zbuf, 0)

    lanes = lax.iota(jnp.int32, 16)
    dump16 = STRIP_ELEMS + lanes

    def _strip_pass(p, carry):
        for j in range(16):
            pltpu.sync_copy(zbuf, strip.at[pl.ds(sid * slice16 + j * 4096, 4096)])
        plsc.subcore_barrier()

        def _scan(k, cnt):
            r16 = rows_v[pl.ds(k * 16, 16)]
            p0 = lax.shift_right_logical(r16, 8)
            p1 = lax.shift_right_logical(r16 + 6, 8)
            m = (p0 == p) | (p1 == p)
            ids = lanes + k * 16
            plsc.store_compressed(sel_v.at[pl.ds(cnt, 16)], ids, mask=m)
            return cnt + plsc.all_reduce_population_count(m)[0]
        cnt = lax.fori_loop(0, SCAN_VREGS, _scan, jnp.int32(0))
        sent = jnp.full((16,), SPT, dtype=jnp.int32)
        for t in range(NG):
            sel_v[pl.ds(cnt + t * 16, 16)] = sent

        def _batch(b, carry):
            s16s = []
            for t in range(NG):
                s16 = sel_v[pl.ds((b * NG + t) * 16, 16)]
                s16s.append(s16)
                gid = jnp.minimum(s16 + wid * SPT, S_TOTAL - 1)
                for mq in range(7):
                    gidx_v[pl.ds(mq * NB + t * 16, 16)] = gid * 7 + mq
            gather = pltpu.async_copy(vals_hbm.at[gidx_v], vals_g, sem)

            @pl.when(b > 0)
            def _():
                pltpu.make_async_copy(st_upd, strip.at[st_idx], sem2).wait()
            gather.wait()

            for t in range(NG):
                s16 = s16s[t]
                r16 = plsc.load_gather(rows_v, [s16])
                c16 = plsc.load_gather(cols_v, [s16])
                ro16 = r16 - p * SR
                base16 = lax.shift_left(ro16, 12) + c16
                xm = [vals_g[pl.ds(mq * NB + t * 16, 16)] for mq in range(7)]
                valids = []
                idxb = []
                for qd in range(7):
                    roq = ro16 + qd
                    valids.append((roq >= 0) & (roq < SR))
                    idxb.append(base16 + qd * N_ORB)
                for q in range(49):
                    k, lq = q // 16, q % 16
                    lqv = jnp.full((16,), lq, dtype=jnp.int32)
                    pr = [xm[mq] * jnp.take_along_axis(
                              c2_v[pl.ds(mq * 64 + k * 16, 16)], lqv, axis=0,
                              mode="promise_in_bounds")
                          for mq in range(7)]
                    acc = ((pr[0] + pr[1]) + (pr[2] + pr[3])) + (
                        (pr[4] + pr[5]) + pr[6])
                    valid = valids[q // 7]
                    idxq = idxb[q // 7] + (q % 7)
                    off = (t * 49 + q) * 16
                    st_idx[pl.ds(off, 16)] = jnp.where(valid, idxq, dump16)
                    st_upd[pl.ds(off, 16)] = jnp.where(valid, acc, 0.0)
            pltpu.async_copy(st_upd, strip.at[st_idx], sem2, add=True)
            return carry
        nb = (cnt + (NB - 1)) // NB
        lax.fori_loop(0, nb, _batch, 0)

        @pl.when(nb > 0)
        def _():
            pltpu.make_async_copy(st_upd, strip.at[st_idx], sem2).wait()
        plsc.subcore_barrier()

        out_off = cid * (NSTRIP * STRIP_ELEMS) + p * STRIP_ELEMS + sid * slice16
        pltpu.sync_copy(strip.at[pl.ds(sid * slice16, slice16)],
                        out_hbm.at[pl.ds(out_off, slice16)])
        plsc.subcore_barrier()
        return carry

    lax.fori_loop(0, NSTRIP, _strip_pass, 0)


def _scatter_partials(vals_flat, c2_flat, rows, cols):
    mesh = plsc.VectorSubcoreMesh(core_axis_name="c", subcore_axis_name="s")
    return pl.kernel(
        _sc_body,
        out_type=jax.ShapeDtypeStruct((2 * NSTRIP * STRIP_ELEMS,), jnp.float32),
        mesh=mesh,
        scratch_types=[
            pltpu.VMEM((SPT + 16,), jnp.int32),
            pltpu.VMEM((SPT + 16,), jnp.int32),
            pltpu.VMEM((448,), jnp.float32),
            pltpu.VMEM((SPT + NB,), jnp.int32),
            pltpu.VMEM((7 * NB,), jnp.int32),
            pltpu.VMEM((7 * NB,), jnp.float32),
            pltpu.VMEM((NG * 49 * 16,), jnp.int32),
            pltpu.VMEM((NG * 49 * 16,), jnp.float32),
            pltpu.VMEM((4096,), jnp.float32),
            pltpu.VMEM_SHARED((STRIP_ELEMS + STRIP_PAD,), jnp.float32),
            pltpu.SemaphoreType.DMA,
            pltpu.SemaphoreType.DMA,
        ],
        compiler_params=pltpu.CompilerParams(needs_layout_passes=False,
                                             use_tc_tiling_on_sc=False),
    )(vals_flat, c2_flat, rows, cols)


def _sym_body(a0, a1, b0, b1, o):
    b = b0[0] + b1[0]
    o[...] = 0.5 * (a0[0] + a1[0] + b.T)


def _symmetrize(A):
    blk = 512
    g = N_ORB // blk
    return pl.pallas_call(
        _sym_body,
        grid=(g, g),
        in_specs=[
            pl.BlockSpec((1, blk, blk), lambda i, j: (0, i, j)),
            pl.BlockSpec((1, blk, blk), lambda i, j: (1, i, j)),
            pl.BlockSpec((1, blk, blk), lambda i, j: (0, j, i)),
            pl.BlockSpec((1, blk, blk), lambda i, j: (1, j, i)),
        ],
        out_specs=pl.BlockSpec((blk, blk), lambda i, j: (i, j)),
        out_shape=jax.ShapeDtypeStruct((N_ORB, N_ORB), jnp.float32),
    )(A, A, A, A)


@jax.jit
def kernel(values, C, rows, cols):
    c2 = jnp.zeros((7, 64), dtype=jnp.float32)
    c2 = c2.at[:, :49].set(jnp.transpose(C, (2, 0, 1)).reshape(7, 49))
    partials = _scatter_partials(values.reshape(-1), c2.reshape(-1),
                                 rows.astype(jnp.int32), cols.astype(jnp.int32))
    return _symmetrize(partials.reshape(2, N_ORB, N_ORB))


# speedup vs baseline: 30.6541x; 1.0068x over previous
import jax
import jax.numpy as jnp
from jax import lax
from jax.experimental import pallas as pl
from jax.experimental.pallas import tpu as pltpu
from jax.experimental.pallas import tpu_sc as plsc

N_ORB = 4096
S_TOTAL = 262144
NW = 32
SPT = S_TOTAL // NW
NSTRIP = 16
SR = N_ORB // NSTRIP
STRIP_ELEMS = SR * N_ORB
STRIP_PAD = 64
NB = 64
NG = NB // 16
SCAN_VREGS = SPT // 16


def _sc_body(vals_hbm, c2_hbm, rows_hbm, cols_hbm, out_hbm,
             rows_v, cols_v, c2_v, sel_v, gidx_v, vals_g, st_idx, st_upd,
             zbuf, strip, sem, sem2):
    cid = lax.axis_index("c")
    sid = lax.axis_index("s")
    wid = cid * 16 + sid
    slice16 = STRIP_ELEMS // 16

    pltpu.sync_copy(rows_hbm.at[pl.ds(wid * SPT, SPT)], rows_v.at[pl.ds(0, SPT)])
    pltpu.sync_copy(cols_hbm.at[pl.ds(wid * SPT, SPT)], cols_v.at[pl.ds(0, SPT)])
    pltpu.sync_copy(c2_hbm, c2_v)
    rows_v[pl.ds(SPT, 16)] = jnp.full((16,), 1 << 20, dtype=jnp.int32)
    cols_v[pl.ds(SPT, 16)] = jnp.zeros((16,), dtype=jnp.int32)

    def _zero_zbuf(i, carry):
        zbuf[pl.ds(i * 16, 16)] = jnp.zeros((16,), dtype=jnp.float32)
        return carry
    lax.fori_loop(0, 4096 // 16, _zero_zbuf, 0)

    lanes = lax.iota(jnp.int32, 16)
    dump16 = STRIP_ELEMS + lanes

    def _strip_pass(p, carry):
        for j in range(16):
            pltpu.sync_copy(zbuf, strip.at[pl.ds(sid * slice16 + j * 4096, 4096)])
        plsc.subcore_barrier()

        def _scan(k, cnt):
            for u in range(4):
                ku = k * 4 + u
                r16 = rows_v[pl.ds(ku * 16, 16)]
                p0 = lax.shift_right_logical(r16, 8)
                p1 = lax.shift_right_logical(r16 + 6, 8)
                m = (p0 == p) | (p1 == p)
                ids = lanes + ku * 16
                plsc.store_compressed(sel_v.at[pl.ds(cnt, 16)], ids, mask=m)
                cnt = cnt + plsc.all_reduce_population_count(m)[0]
            return cnt
        cnt = lax.fori_loop(0, SCAN_VREGS // 4, _scan, jnp.int32(0))
        sent = jnp.full((16,), SPT, dtype=jnp.int32)
        for t in range(NG):
            sel_v[pl.ds(cnt + t * 16, 16)] = sent

        def _batch(b, carry):
            s16s = []
            for t in range(NG):
                s16 = sel_v[pl.ds((b * NG + t) * 16, 16)]
                s16s.append(s16)
                gid = jnp.minimum(s16 + wid * SPT, S_TOTAL - 1)
                for mq in range(7):
                    gidx_v[pl.ds(mq * NB + t * 16, 16)] = gid * 7 + mq
            gather = pltpu.async_copy(vals_hbm.at[gidx_v], vals_g, sem)

            @pl.when(b > 0)
            def _():
                pltpu.make_async_copy(st_upd, strip.at[st_idx], sem2).wait()
            gather.wait()

            for t in range(NG):
                s16 = s16s[t]
                r16 = plsc.load_gather(rows_v, [s16])
                c16 = plsc.load_gather(cols_v, [s16])
                ro16 = r16 - p * SR
                base16 = lax.shift_left(ro16, 12) + c16
                xm = [vals_g[pl.ds(mq * NB + t * 16, 16)] for mq in range(7)]
                valids = []
                idxb = []
                for qd in range(7):
                    roq = ro16 + qd
                    valids.append((roq >= 0) & (roq < SR))
                    idxb.append(base16 + qd * N_ORB)
                for q in range(49):
                    k, lq = q // 16, q % 16
                    lqv = jnp.full((16,), lq, dtype=jnp.int32)
                    pr = [xm[mq] * jnp.take_along_axis(
                              c2_v[pl.ds(mq * 64 + k * 16, 16)], lqv, axis=0,
                              mode="promise_in_bounds")
                          for mq in range(7)]
                    acc = ((pr[0] + pr[1]) + (pr[2] + pr[3])) + (
                        (pr[4] + pr[5]) + pr[6])
                    valid = valids[q // 7]
                    idxq = idxb[q // 7] + (q % 7)
                    off = (t * 49 + q) * 16
                    st_idx[pl.ds(off, 16)] = jnp.where(valid, idxq, dump16)
                    st_upd[pl.ds(off, 16)] = jnp.where(valid, acc, 0.0)
            pltpu.async_copy(st_upd, strip.at[st_idx], sem2, add=True)
            return carry
        nb = (cnt + (NB - 1)) // NB
        lax.fori_loop(0, nb, _batch, 0)

        @pl.when(nb > 0)
        def _():
            pltpu.make_async_copy(st_upd, strip.at[st_idx], sem2).wait()
        plsc.subcore_barrier()

        out_off = cid * (NSTRIP * STRIP_ELEMS) + p * STRIP_ELEMS + sid * slice16
        pltpu.sync_copy(strip.at[pl.ds(sid * slice16, slice16)],
                        out_hbm.at[pl.ds(out_off, slice16)])
        plsc.subcore_barrier()
        return carry

    lax.fori_loop(0, NSTRIP, _strip_pass, 0)


def _scatter_partials(vals_flat, c2_flat, rows, cols):
    mesh = plsc.VectorSubcoreMesh(core_axis_name="c", subcore_axis_name="s")
    return pl.kernel(
        _sc_body,
        out_type=jax.ShapeDtypeStruct((2 * NSTRIP * STRIP_ELEMS,), jnp.float32),
        mesh=mesh,
        scratch_types=[
            pltpu.VMEM((SPT + 16,), jnp.int32),
            pltpu.VMEM((SPT + 16,), jnp.int32),
            pltpu.VMEM((448,), jnp.float32),
            pltpu.VMEM((SPT + NB,), jnp.int32),
            pltpu.VMEM((7 * NB,), jnp.int32),
            pltpu.VMEM((7 * NB,), jnp.float32),
            pltpu.VMEM((NG * 49 * 16,), jnp.int32),
            pltpu.VMEM((NG * 49 * 16,), jnp.float32),
            pltpu.VMEM((4096,), jnp.float32),
            pltpu.VMEM_SHARED((STRIP_ELEMS + STRIP_PAD,), jnp.float32),
            pltpu.SemaphoreType.DMA,
            pltpu.SemaphoreType.DMA,
        ],
        compiler_params=pltpu.CompilerParams(needs_layout_passes=False,
                                             use_tc_tiling_on_sc=False),
    )(vals_flat, c2_flat, rows, cols)


def _sym_body(a0, a1, b0, b1, o):
    b = b0[0] + b1[0]
    o[...] = 0.5 * (a0[0] + a1[0] + b.T)


def _symmetrize(A):
    blk = 512
    g = N_ORB // blk
    return pl.pallas_call(
        _sym_body,
        grid=(g, g),
        in_specs=[
            pl.BlockSpec((1, blk, blk), lambda i, j: (0, i, j)),
            pl.BlockSpec((1, blk, blk), lambda i, j: (1, i, j)),
            pl.BlockSpec((1, blk, blk), lambda i, j: (0, j, i)),
            pl.BlockSpec((1, blk, blk), lambda i, j: (1, j, i)),
        ],
        out_specs=pl.BlockSpec((blk, blk), lambda i, j: (i, j)),
        out_shape=jax.ShapeDtypeStruct((N_ORB, N_ORB), jnp.float32),
    )(A, A, A, A)


@jax.jit
def kernel(values, C, rows, cols):
    c2 = jnp.zeros((7, 64), dtype=jnp.float32)
    c2 = c2.at[:, :49].set(jnp.transpose(C, (2, 0, 1)).reshape(7, 49))
    partials = _scatter_partials(values.reshape(-1), c2.reshape(-1),
                                 rows.astype(jnp.int32), cols.astype(jnp.int32))
    return _symmetrize(partials.reshape(2, N_ORB, N_ORB))
